# baseline (device time: 24063 ns/iter reference)
import jax
import jax.numpy as jnp
from jax import lax
from jax.experimental import pallas as pl
from jax.experimental.pallas import tpu as pltpu

N_DEV = 4
BLK = 64
NCH = 4


def kernel(x):
    m, n = x.shape
    nblk = m // BLK
    bpc = nblk // NCH
    x3 = x.reshape(nblk, BLK, n)

    def body(
        x_ref,
        out_ref,
        buf,
        gather_ref,
        in_sems,
        out_sems,
        send_sems,
        recv_sems,
        ack_sem,
    ):
        my = lax.axis_index("i")

        in_copies = []
        for c in range(NCH):
            cp = pltpu.make_async_copy(
                x_ref.at[pl.ds(c * bpc, bpc)],
                buf.at[pl.ds(c * bpc, bpc)],
                in_sems.at[c],
            )
            cp.start()
            in_copies.append(cp)

        barrier_sem = pltpu.get_barrier_semaphore()
        for k in range(1, N_DEV):
            pl.semaphore_signal(
                barrier_sem,
                inc=1,
                device_id=((my + k) % N_DEV,),
                device_id_type=pl.DeviceIdType.MESH,
            )
        pl.semaphore_wait(barrier_sem, N_DEV - 1)

        for c in range(NCH):
            in_copies[c].wait()
            r0 = c * bpc
            s = 1
            while s < BLK:
                buf[pl.ds(r0, bpc), pl.ds(s, BLK - s), :] = (
                    buf[pl.ds(r0, bpc), pl.ds(s, BLK - s), :]
                    * buf[pl.ds(r0, bpc), pl.ds(0, BLK - s), :]
                )
                s *= 2

        ct = buf[:, BLK - 1, :]
        s = 1
        while s < nblk:
            shifted = jnp.concatenate(
                [jnp.ones((s, n), jnp.float32), ct[: nblk - s, :]], axis=0
            )
            ct = ct * shifted
            s *= 2

        gather_ref[0, :] = ct[nblk - 1]
        copies = []
        for k in range(1, N_DEV):
            rdma = pltpu.make_async_remote_copy(
                src_ref=gather_ref.at[0],
                dst_ref=gather_ref.at[k],
                send_sem=send_sems.at[k - 1],
                recv_sem=recv_sems.at[k - 1],
                device_id=((my + k) % N_DEV,),
                device_id_type=pl.DeviceIdType.MESH,
            )
            rdma.start()
            copies.append(rdma)

        bpfx = jnp.concatenate(
            [jnp.ones((1, n), jnp.float32), ct[: nblk - 1, :]], axis=0
        )

        for rdma in copies:
            rdma.wait_send()
            rdma.wait_recv()
        g = gather_ref[...]

        for k in range(1, N_DEV):
            pl.semaphore_signal(
                ack_sem,
                inc=1,
                device_id=((my + k) % N_DEV,),
                device_id_type=pl.DeviceIdType.MESH,
            )

        ones = jnp.ones((n,), jnp.float32)
        pfx = ones
        for k in range(1, N_DEV):
            pfx = pfx * jnp.where(my >= k, g[k], ones)
        bpfx2 = bpfx * pfx[None, :]

        out_copies = []
        for c in range(NCH):
            r0 = c * bpc
            buf[pl.ds(r0, bpc), :, :] = (
                buf[pl.ds(r0, bpc), :, :]
                * bpfx2[r0 : r0 + bpc, :][:, None, :]
            )
            cp = pltpu.make_async_copy(
                buf.at[pl.ds(r0, bpc)],
                out_ref.at[pl.ds(r0, bpc)],
                out_sems.at[c],
            )
            cp.start()
            out_copies.append(cp)
        for cp in out_copies:
            cp.wait()

        pl.semaphore_wait(ack_sem, N_DEV - 1)

    out3 = pl.pallas_call(
        body,
        out_shape=jax.ShapeDtypeStruct((nblk, BLK, n), jnp.float32),
        in_specs=[pl.BlockSpec(memory_space=pl.ANY)],
        out_specs=pl.BlockSpec(memory_space=pltpu.MemorySpace.HBM),
        scratch_shapes=[
            pltpu.VMEM((nblk, BLK, n), jnp.float32),
            pltpu.VMEM((N_DEV, n), jnp.float32),
            pltpu.SemaphoreType.DMA((NCH,)),
            pltpu.SemaphoreType.DMA((NCH,)),
            pltpu.SemaphoreType.DMA((N_DEV - 1,)),
            pltpu.SemaphoreType.DMA((N_DEV - 1,)),
            pltpu.SemaphoreType.REGULAR,
        ],
        compiler_params=pltpu.CompilerParams(collective_id=0),
    )(x3)
    return out3.reshape(m, n)


# device time: 23972 ns/iter; 1.0038x vs baseline; 1.0038x over previous
import jax
import jax.numpy as jnp
from jax import lax
from jax.experimental import pallas as pl
from jax.experimental.pallas import tpu as pltpu

N_DEV = 4
BLK = 64
NCH = 4


def kernel(x):
    m, n = x.shape
    nblk = m // BLK
    bpc = nblk // NCH
    x3 = x.reshape(nblk, BLK, n)

    def body(
        x_ref,
        out_ref,
        buf,
        gather_ref,
        in_sems,
        out_sems,
        send_sems,
        recv_sems,
        ack_sem,
    ):
        my = lax.axis_index("i")

        in_copies = []
        for c in range(NCH):
            cp = pltpu.make_async_copy(
                x_ref.at[pl.ds(c * bpc, bpc)],
                buf.at[pl.ds(c * bpc, bpc)],
                in_sems.at[c],
            )
            cp.start()
            in_copies.append(cp)

        barrier_sem = pltpu.get_barrier_semaphore()
        for k in range(1, N_DEV):
            pl.semaphore_signal(
                barrier_sem,
                inc=1,
                device_id=((my + k) % N_DEV,),
                device_id_type=pl.DeviceIdType.MESH,
            )
        pl.semaphore_wait(barrier_sem, N_DEV - 1)

        for c in range(NCH):
            in_copies[c].wait()
            r0 = c * bpc
            s = 1
            while s < BLK:
                buf[pl.ds(r0, bpc), pl.ds(s, BLK - s), :] = (
                    buf[pl.ds(r0, bpc), pl.ds(s, BLK - s), :]
                    * buf[pl.ds(r0, bpc), pl.ds(0, BLK - s), :]
                )
                s *= 2

        ct = buf[:, BLK - 1, :]
        t = ct
        h = nblk
        while h > 1:
            h //= 2
            t = t[:h, :] * t[h:, :]
        gather_ref[0, :] = t[0]
        copies = []
        for k in range(1, N_DEV):
            rdma = pltpu.make_async_remote_copy(
                src_ref=gather_ref.at[0],
                dst_ref=gather_ref.at[k],
                send_sem=send_sems.at[k - 1],
                recv_sem=recv_sems.at[k - 1],
                device_id=((my + k) % N_DEV,),
                device_id_type=pl.DeviceIdType.MESH,
            )
            rdma.start()
            copies.append(rdma)

        s = 1
        while s < nblk:
            shifted = jnp.concatenate(
                [jnp.ones((s, n), jnp.float32), ct[: nblk - s, :]], axis=0
            )
            ct = ct * shifted
            s *= 2
        bpfx = jnp.concatenate(
            [jnp.ones((1, n), jnp.float32), ct[: nblk - 1, :]], axis=0
        )

        for rdma in copies:
            rdma.wait_send()
            rdma.wait_recv()
        g = gather_ref[...]

        for k in range(1, N_DEV):
            pl.semaphore_signal(
                ack_sem,
                inc=1,
                device_id=((my + k) % N_DEV,),
                device_id_type=pl.DeviceIdType.MESH,
            )

        ones = jnp.ones((n,), jnp.float32)
        pfx = ones
        for k in range(1, N_DEV):
            pfx = pfx * jnp.where(my >= k, g[k], ones)
        bpfx2 = bpfx * pfx[None, :]

        out_copies = []
        for c in range(NCH):
            r0 = c * bpc
            buf[pl.ds(r0, bpc), :, :] = (
                buf[pl.ds(r0, bpc), :, :]
                * bpfx2[r0 : r0 + bpc, :][:, None, :]
            )
            cp = pltpu.make_async_copy(
                buf.at[pl.ds(r0, bpc)],
                out_ref.at[pl.ds(r0, bpc)],
                out_sems.at[c],
            )
            cp.start()
            out_copies.append(cp)
        for cp in out_copies:
            cp.wait()

        pl.semaphore_wait(ack_sem, N_DEV - 1)

    out3 = pl.pallas_call(
        body,
        out_shape=jax.ShapeDtypeStruct((nblk, BLK, n), jnp.float32),
        in_specs=[pl.BlockSpec(memory_space=pl.ANY)],
        out_specs=pl.BlockSpec(memory_space=pltpu.MemorySpace.HBM),
        scratch_shapes=[
            pltpu.VMEM((nblk, BLK, n), jnp.float32),
            pltpu.VMEM((N_DEV, n), jnp.float32),
            pltpu.SemaphoreType.DMA((NCH,)),
            pltpu.SemaphoreType.DMA((NCH,)),
            pltpu.SemaphoreType.DMA((N_DEV - 1,)),
            pltpu.SemaphoreType.DMA((N_DEV - 1,)),
            pltpu.SemaphoreType.REGULAR,
        ],
        compiler_params=pltpu.CompilerParams(collective_id=0),
    )(x3)
    return out3.reshape(m, n)
